# baseline (speedup 1.0000x reference)
import functools

import jax
import jax.numpy as jnp
from jax import lax
from jax.experimental import pallas as pl
from jax.experimental.pallas import tpu as pltpu
from jax.experimental.pallas import tpu_sc as plsc

N = 4096
KB = 512
NKB = N // KB
M_BLK = 512
G_ROWS = N + KB
G_WORDS = G_ROWS * N
TRASH = N * N
GBLK = KB * N
NC = 2
NS = 16
CPT = N // (NC * NS)


def _meta_kernel(idx_ref, meta_ref):
    idx = idx_ref[...]
    tap_min = jnp.maximum(jnp.floor(jnp.min(idx)).astype(jnp.int32) - 1, 0)
    tap_max = jnp.minimum(jnp.floor(jnp.max(idx)).astype(jnp.int32) + 2, N - 1)
    base = tap_min >> 9
    nblk = (tap_max >> 9) - base + 1
    lane = jax.lax.broadcasted_iota(jnp.int32, (1, 128), 1)
    meta_ref[...] = jnp.where(lane == 0, base, jnp.where(lane == 1, nblk, 0))


def _zero_kernel(meta_ref, g_ref):
    @pl.when(pl.program_id(0) < meta_ref[1])
    def _():
        g_ref[...] = jnp.zeros((GBLK,), jnp.float32)


def _lane_shuffle(v, shift):
    lane = lax.broadcasted_iota(jnp.int32, (16,), 0)
    return lax.gather(
        v, (lane ^ shift)[:, None],
        dimension_numbers=lax.GatherDimensionNumbers(
            offset_dims=(), collapsed_slice_dims=(0,), start_index_map=(0,)),
        slice_sizes=(1,),
        mode=lax.GatherScatterMode.PROMISE_IN_BOUNDS)


@functools.partial(
    pl.kernel,
    mesh=plsc.VectorSubcoreMesh(core_axis_name="c", subcore_axis_name="s"),
    out_type=[],
    scratch_types=[
        pltpu.VMEM((N,), jnp.float32),
        pltpu.VMEM((4, CPT), jnp.int32),
        pltpu.VMEM((4, CPT), jnp.float32),
        pltpu.SemaphoreType.DMA,
    ],
)
def _sc_scatter(idx_hbm, g_ref, idx_v, ibuf, vbuf, sem):
    c = lax.axis_index("c")
    s = lax.axis_index("s")

    pltpu.sync_copy(idx_hbm, idx_v)

    def scan_body(g, mn):
        return jnp.minimum(mn, idx_v[pl.ds(g * 16, 16)])

    mn_v = lax.fori_loop(1, N // 16, scan_body, idx_v[pl.ds(0, 16)])
    tapmin_v = jnp.maximum(mn_v.astype(jnp.int32) - 1, 0)
    for shift in (8, 4, 2, 1):
        tapmin_v = jnp.minimum(tapmin_v, _lane_shuffle(tapmin_v, shift))
    base_col = (tapmin_v >> 9) * KB

    j0 = c * (N // NC) + s * CPT
    lane = lax.broadcasted_iota(jnp.int32, (16,), 0)
    for g in range(CPT // 16):
        x = idx_v[pl.ds(j0 + g * 16, 16)]
        iv = x.astype(jnp.int32)
        f = x - iv.astype(jnp.float32)
        f2 = f * f
        f3 = f2 * f
        w0 = -0.5 * f + f2 - 0.5 * f3
        w1 = 1.0 - 2.5 * f2 + 1.5 * f3
        w2 = 0.5 * f + 2.0 * f2 - 1.5 * f3
        w3 = -0.5 * f2 + 0.5 * f3
        jv = j0 + g * 16 + lane
        t0 = jnp.clip(iv - 1, 0, N - 1)
        t1 = jnp.clip(iv, 0, N - 1)
        t2 = jnp.clip(iv + 1, 0, N - 1)
        t3 = jnp.clip(iv + 2, 0, N - 1)
        lo_dup = t0 == t1
        w1 = w1 + jnp.where(lo_dup, w0, 0.0)
        a0 = jnp.where(lo_dup, TRASH + jv, (t0 - base_col) * N + jv)
        hi_dup = t3 == t2
        w2 = w2 + jnp.where(hi_dup, w3, 0.0)
        a3 = jnp.where(hi_dup, TRASH + jv, (t3 - base_col) * N + jv)
        a1 = (t1 - base_col) * N + jv
        a2 = (t2 - base_col) * N + jv
        for t, (a, w) in enumerate(((a0, w0), (a1, w1), (a2, w2), (a3, w3))):
            ibuf[t, pl.ds(g * 16, 16)] = a
            vbuf[t, pl.ds(g * 16, 16)] = w
    for t in range(4):
        pltpu.async_copy(vbuf.at[t], g_ref.at[ibuf.at[t]], sem)
    for t in range(4):
        pltpu.make_async_copy(vbuf.at[t], g_ref.at[ibuf.at[t]], sem).wait()


def _matmul_kernel(meta_ref, a_ref, g_ref, o_ref, acc_ref, gbf_ref):
    m = pl.program_id(0)
    k = pl.program_id(1)
    nblk = meta_ref[1]

    @pl.when((k < nblk) & ((nblk > 1) | ((m == 0) & (k == 0))))
    def _():
        gbf_ref[...] = jnp.reshape(g_ref[...], (KB, N)).astype(jnp.bfloat16)

    @pl.when(k == 0)
    def _():
        acc_ref[...] = jnp.dot(a_ref[...].astype(jnp.bfloat16), gbf_ref[...],
                               preferred_element_type=jnp.float32)

    @pl.when((k > 0) & (k < nblk))
    def _():
        acc_ref[...] += jnp.dot(a_ref[...].astype(jnp.bfloat16), gbf_ref[...],
                                preferred_element_type=jnp.float32)

    @pl.when(k == NKB - 1)
    def _():
        o_ref[...] = acc_ref[...]


def kernel(src, indices):
    idx2d = indices.reshape(1, N)

    meta = pl.pallas_call(
        _meta_kernel,
        in_specs=[pl.BlockSpec((1, N), lambda: (0, 0))],
        out_specs=pl.BlockSpec((1, 128), lambda: (0, 0)),
        out_shape=jax.ShapeDtypeStruct((1, 128), jnp.int32),
    )(idx2d)
    meta1d = meta.reshape(128)

    def kk(k, meta_ref):
        return jnp.minimum(k, meta_ref[1] - 1)

    g0 = pl.pallas_call(
        _zero_kernel,
        grid_spec=pltpu.PrefetchScalarGridSpec(
            num_scalar_prefetch=1,
            grid=(NKB,),
            in_specs=[],
            out_specs=pl.BlockSpec((GBLK,), lambda k, meta: (kk(k, meta),)),
        ),
        out_shape=jax.ShapeDtypeStruct((G_WORDS,), jnp.float32),
        compiler_params=pltpu.CompilerParams(
            dimension_semantics=("arbitrary",),
        ),
    )(meta1d)

    g_ref = jax.new_ref(g0)
    _sc_scatter(indices, g_ref)
    g = g_ref[...]

    out = pl.pallas_call(
        _matmul_kernel,
        grid_spec=pltpu.PrefetchScalarGridSpec(
            num_scalar_prefetch=1,
            grid=(N // M_BLK, NKB),
            in_specs=[
                pl.BlockSpec((M_BLK, KB),
                             lambda m, k, meta: (m, meta[0] + kk(k, meta))),
                pl.BlockSpec((GBLK,),
                             lambda m, k, meta: (kk(k, meta),)),
            ],
            out_specs=pl.BlockSpec((M_BLK, N), lambda m, k, meta: (m, 0)),
            scratch_shapes=[
                pltpu.VMEM((M_BLK, N), jnp.float32),
                pltpu.VMEM((KB, N), jnp.bfloat16),
            ],
        ),
        out_shape=jax.ShapeDtypeStruct((N, N), jnp.float32),
        compiler_params=pltpu.CompilerParams(
            dimension_semantics=("arbitrary", "arbitrary"),
        ),
    )(meta1d, src, g)
    return out


# speedup vs baseline: 2.2913x; 2.2913x over previous
import jax
import jax.numpy as jnp
from jax.experimental import pallas as pl
from jax.experimental.pallas import tpu as pltpu

N = 4096
KB = 512
NKB = N // KB
M_BLK = 512


def _build_g_kernel(idx_ref, g_ref, meta_ref, sm_ref):
    k = pl.program_id(0)

    @pl.when(k == 0)
    def _():
        idx = idx_ref[...]
        tap_min = jnp.floor(jnp.min(idx)).astype(jnp.int32) - 1
        tap_min = jnp.maximum(tap_min, 0)
        tap_max = jnp.floor(jnp.max(idx)).astype(jnp.int32) + 2
        tap_max = jnp.minimum(tap_max, N - 1)
        base = tap_min >> 9
        nblk = (tap_max >> 9) - base + 1
        sm_ref[0] = base
        sm_ref[1] = nblk
        lane = jax.lax.broadcasted_iota(jnp.int32, (1, 128), 1)
        meta_ref[...] = jnp.where(lane == 0, base, jnp.where(lane == 1, nblk, 0))

    base = sm_ref[0]
    nblk = sm_ref[1]

    @pl.when(k < nblk)
    def _():
        idx = idx_ref[...]
        i = jnp.floor(idx)
        f = idx - i
        ii = i.astype(jnp.int32)
        f2 = f * f
        f3 = f2 * f
        w0 = -0.5 * f + f2 - 0.5 * f3
        w1 = 1.0 - 2.5 * f2 + 1.5 * f3
        w2 = 0.5 * f + 2.0 * f2 - 1.5 * f3
        w3 = -0.5 * f2 + 0.5 * f3
        c0 = (base + k) * KB
        c = jax.lax.broadcasted_iota(jnp.int32, (KB, N), 0) + c0
        g = jnp.zeros((KB, N), jnp.float32)
        for t, w in ((-1, w0), (0, w1), (1, w2), (2, w3)):
            tap = jnp.clip(ii + t, 0, N - 1)
            g = g + jnp.where(c == tap, w, 0.0)
        g_ref[...] = g.astype(jnp.bfloat16)


def _matmul_kernel(meta_ref, a_ref, g_ref, o_ref, acc_ref):
    k = pl.program_id(1)
    nblk = meta_ref[1]

    @pl.when(k == 0)
    def _():
        acc_ref[...] = jnp.dot(a_ref[...].astype(jnp.bfloat16), g_ref[...],
                               preferred_element_type=jnp.float32)

    @pl.when((k > 0) & (k < nblk))
    def _():
        acc_ref[...] += jnp.dot(a_ref[...].astype(jnp.bfloat16), g_ref[...],
                                preferred_element_type=jnp.float32)

    @pl.when(k == NKB - 1)
    def _():
        o_ref[...] = acc_ref[...]


def kernel(src, indices):
    idx2d = indices.reshape(1, N)

    def gb_out_map(k):
        return (k, 0)

    g, meta = pl.pallas_call(
        _build_g_kernel,
        grid=(NKB,),
        in_specs=[pl.BlockSpec((1, N), lambda k: (0, 0))],
        out_specs=[
            pl.BlockSpec((KB, N), gb_out_map),
            pl.BlockSpec((1, 128), lambda k: (0, 0)),
        ],
        out_shape=[
            jax.ShapeDtypeStruct((N, N), jnp.bfloat16),
            jax.ShapeDtypeStruct((1, 128), jnp.int32),
        ],
        scratch_shapes=[pltpu.SMEM((2,), jnp.int32)],
        compiler_params=pltpu.CompilerParams(
            dimension_semantics=("arbitrary",),
        ),
    )(idx2d)

    meta1d = meta.reshape(128)

    def kk(k, meta_ref):
        return jnp.minimum(k, meta_ref[1] - 1)

    out = pl.pallas_call(
        _matmul_kernel,
        grid_spec=pltpu.PrefetchScalarGridSpec(
            num_scalar_prefetch=1,
            grid=(N // M_BLK, NKB),
            in_specs=[
                pl.BlockSpec((M_BLK, KB),
                             lambda m, k, meta: (m, meta[0] + kk(k, meta))),
                pl.BlockSpec((KB, N),
                             lambda m, k, meta: (kk(k, meta), 0)),
            ],
            out_specs=pl.BlockSpec((M_BLK, N), lambda m, k, meta: (m, 0)),
            scratch_shapes=[pltpu.VMEM((M_BLK, N), jnp.float32)],
        ),
        out_shape=jax.ShapeDtypeStruct((N, N), jnp.float32),
        compiler_params=pltpu.CompilerParams(
            dimension_semantics=("arbitrary", "arbitrary"),
        ),
    )(meta1d, src, g)
    return out


# speedup vs baseline: 2.4861x; 1.0850x over previous
import jax
import jax.numpy as jnp
from jax.experimental import pallas as pl
from jax.experimental.pallas import tpu as pltpu

N = 4096
KB = 512
NKB = N // KB
M_BLK = 512


def _meta_kernel(idx_ref, meta_ref):
    idx = idx_ref[...]
    tap_min = jnp.maximum(jnp.floor(jnp.min(idx)).astype(jnp.int32) - 1, 0)
    tap_max = jnp.minimum(jnp.floor(jnp.max(idx)).astype(jnp.int32) + 2, N - 1)
    base = tap_min >> 9
    nblk = (tap_max >> 9) - base + 1
    lane = jax.lax.broadcasted_iota(jnp.int32, (1, 128), 1)
    meta_ref[...] = jnp.where(lane == 0, base, jnp.where(lane == 1, nblk, 0))


def _main_kernel(meta_ref, idx_ref, a_ref, o_ref, g_ref):
    m = pl.program_id(0)
    k = pl.program_id(1)
    base = meta_ref[0]
    nblk = meta_ref[1]

    @pl.when((m == 0) & (k < nblk))
    def _():
        idx = idx_ref[...]
        i = jnp.floor(idx)
        f = idx - i
        ii = i.astype(jnp.int32)
        f2 = f * f
        f3 = f2 * f
        w0 = -0.5 * f + f2 - 0.5 * f3
        w1 = 1.0 - 2.5 * f2 + 1.5 * f3
        w2 = 0.5 * f + 2.0 * f2 - 1.5 * f3
        w3 = -0.5 * f2 + 0.5 * f3
        c0 = (base + k) * KB
        c = jax.lax.broadcasted_iota(jnp.int32, (KB, N), 0) + c0
        g = jnp.zeros((KB, N), jnp.float32)
        for t, w in ((-1, w0), (0, w1), (1, w2), (2, w3)):
            tap = jnp.clip(ii + t, 0, N - 1)
            g = g + jnp.where(c == tap, w, 0.0)
        g_ref[pl.ds(k * KB, KB), :] = g.astype(jnp.bfloat16)

    @pl.when(k == 0)
    def _():
        o_ref[...] = jnp.dot(a_ref[...].astype(jnp.bfloat16),
                             g_ref[pl.ds(0, KB), :],
                             preferred_element_type=jnp.float32)

    @pl.when((k > 0) & (k < nblk))
    def _():
        o_ref[...] += jnp.dot(a_ref[...].astype(jnp.bfloat16),
                              g_ref[pl.ds(k * KB, KB), :],
                              preferred_element_type=jnp.float32)


def kernel(src, indices):
    idx2d = indices.reshape(1, N)

    meta = pl.pallas_call(
        _meta_kernel,
        in_specs=[pl.BlockSpec((1, N), lambda: (0, 0))],
        out_specs=pl.BlockSpec((1, 128), lambda: (0, 0)),
        out_shape=jax.ShapeDtypeStruct((1, 128), jnp.int32),
    )(idx2d)
    meta1d = meta.reshape(128)

    def kk(k, meta_ref):
        return jnp.minimum(k, meta_ref[1] - 1)

    out = pl.pallas_call(
        _main_kernel,
        grid_spec=pltpu.PrefetchScalarGridSpec(
            num_scalar_prefetch=1,
            grid=(N // M_BLK, NKB),
            in_specs=[
                pl.BlockSpec((1, N), lambda m, k, meta: (0, 0)),
                pl.BlockSpec((M_BLK, KB),
                             lambda m, k, meta: (m, meta[0] + kk(k, meta))),
            ],
            out_specs=pl.BlockSpec((M_BLK, N), lambda m, k, meta: (m, 0)),
            scratch_shapes=[
                pltpu.VMEM((N, N), jnp.bfloat16),
            ],
        ),
        out_shape=jax.ShapeDtypeStruct((N, N), jnp.float32),
        compiler_params=pltpu.CompilerParams(
            dimension_semantics=("arbitrary", "arbitrary"),
        ),
    )(meta1d, idx2d, src)
    return out


# speedup vs baseline: 3.6931x; 1.4855x over previous
import jax
import jax.numpy as jnp
from jax.experimental import pallas as pl
from jax.experimental.pallas import tpu as pltpu

N = 4096
KB = 512
NKB = N // KB
M_BLK = 512


def _meta_kernel(idx_ref, meta_ref):
    idx = idx_ref[...]
    tap_min = jnp.maximum(jnp.floor(jnp.min(idx)).astype(jnp.int32) - 1, 0)
    tap_max = jnp.minimum(jnp.floor(jnp.max(idx)).astype(jnp.int32) + 2, N - 1)
    base = tap_min >> 9
    nblk = (tap_max >> 9) - base + 1
    lane = jax.lax.broadcasted_iota(jnp.int32, (1, 128), 1)
    meta_ref[...] = jnp.where(lane == 0, base, jnp.where(lane == 1, nblk, 0))


def _main_kernel(meta_ref, idx_ref, a_ref, src_hbm, o_ref, g_ref, xbuf, sem):
    m = pl.program_id(0)
    base = meta_ref[0]
    nblk = meta_ref[1]

    @pl.when(m == 0)
    def _():
        idx = idx_ref[...]
        i = jnp.floor(idx)
        f = idx - i
        ii = i.astype(jnp.int32)
        f2 = f * f
        f3 = f2 * f
        w0 = -0.5 * f + f2 - 0.5 * f3
        w1 = 1.0 - 2.5 * f2 + 1.5 * f3
        w2 = 0.5 * f + 2.0 * f2 - 1.5 * f3
        w3 = -0.5 * f2 + 0.5 * f3
        for k in range(NKB):
            @pl.when(k < nblk)
            def _():
                c0 = (base + k) * KB
                c = jax.lax.broadcasted_iota(jnp.int32, (KB, N), 0) + c0
                g = jnp.zeros((KB, N), jnp.float32)
                for t, w in ((-1, w0), (0, w1), (1, w2), (2, w3)):
                    tap = jnp.clip(ii + t, 0, N - 1)
                    g = g + jnp.where(c == tap, w, 0.0)
                g_ref[pl.ds(k * KB, KB), :] = g.astype(jnp.bfloat16)

    o_ref[...] = jnp.dot(a_ref[...].astype(jnp.bfloat16),
                         g_ref[pl.ds(0, KB), :],
                         preferred_element_type=jnp.float32)

    def body(k, _):
        copy = pltpu.make_async_copy(
            src_hbm.at[pl.ds(m * M_BLK, M_BLK), pl.ds((base + k) * KB, KB)],
            xbuf, sem)
        copy.start()
        copy.wait()
        o_ref[...] += jnp.dot(xbuf[...].astype(jnp.bfloat16),
                              g_ref[pl.ds(k * KB, KB), :],
                              preferred_element_type=jnp.float32)
        return 0

    jax.lax.fori_loop(1, nblk, body, 0)


def kernel(src, indices):
    idx2d = indices.reshape(1, N)

    meta = pl.pallas_call(
        _meta_kernel,
        in_specs=[pl.BlockSpec((1, N), lambda: (0, 0))],
        out_specs=pl.BlockSpec((1, 128), lambda: (0, 0)),
        out_shape=jax.ShapeDtypeStruct((1, 128), jnp.int32),
    )(idx2d)
    meta1d = meta.reshape(128)

    out = pl.pallas_call(
        _main_kernel,
        grid_spec=pltpu.PrefetchScalarGridSpec(
            num_scalar_prefetch=1,
            grid=(N // M_BLK,),
            in_specs=[
                pl.BlockSpec((1, N), lambda m, meta: (0, 0)),
                pl.BlockSpec((M_BLK, KB), lambda m, meta: (m, meta[0])),
                pl.BlockSpec(memory_space=pl.ANY),
            ],
            out_specs=pl.BlockSpec((M_BLK, N), lambda m, meta: (m, 0)),
            scratch_shapes=[
                pltpu.VMEM((N, N), jnp.bfloat16),
                pltpu.VMEM((M_BLK, KB), jnp.float32),
                pltpu.SemaphoreType.DMA,
            ],
        ),
        out_shape=jax.ShapeDtypeStruct((N, N), jnp.float32),
        compiler_params=pltpu.CompilerParams(
            dimension_semantics=("arbitrary",),
        ),
    )(meta1d, idx2d, src, src)
    return out
